# Initial kernel scaffold; baseline (speedup 1.0000x reference)
import jax
import jax.numpy as jnp
from jax.experimental import pallas as pl


def kernel(sents, entdists, numdists, word_table, ent_table, num_table):
    raise NotImplementedError("write your pallas kernel here")


# speedup vs baseline: 7.3482x; 7.3482x over previous
import functools

import jax
import jax.numpy as jnp
from jax import lax
from jax.experimental import pallas as pl
from jax.experimental.pallas import tpu as pltpu
from jax.experimental.pallas import tpu_sc as plsc

WORD_DIM = 64
DIST_DIM = 32
OUT_DIM = WORD_DIM + 2 * DIST_DIM
BLK = 128


@functools.lru_cache(maxsize=None)
def _make_sc_kernel(N: int):
    info = plsc.get_sparse_core_info()
    NC, NS = info.num_cores, info.num_subcores
    NW = NC * NS
    assert N % (NW * BLK) == 0
    chunk = N // NW
    nblk = chunk // BLK
    mesh = plsc.VectorSubcoreMesh(core_axis_name="c", subcore_axis_name="s")

    @functools.partial(
        pl.kernel,
        mesh=mesh,
        out_type=jax.ShapeDtypeStruct((N, OUT_DIM), jnp.float32),
        compiler_params=pltpu.CompilerParams(use_tc_tiling_on_sc=False),
        scratch_types=[
            pltpu.VMEM((chunk,), jnp.int32),
            pltpu.VMEM((chunk,), jnp.int32),
            pltpu.VMEM((chunk,), jnp.int32),
            pltpu.VMEM((BLK, WORD_DIM), jnp.float32),
            pltpu.VMEM((BLK, DIST_DIM), jnp.float32),
            pltpu.VMEM((BLK, DIST_DIM), jnp.float32),
            pltpu.SemaphoreType.DMA,
        ],
    )
    def k(widx_hbm, eidx_hbm, nidx_hbm, wtab, etab, ntab, out_hbm,
          widx_v, eidx_v, nidx_v, wbuf, ebuf, nbuf, gsem):
        wid = lax.axis_index("s") * NC + lax.axis_index("c")
        base = wid * chunk
        pltpu.sync_copy(widx_hbm.at[pl.ds(base, chunk)], widx_v)
        pltpu.sync_copy(eidx_hbm.at[pl.ds(base, chunk)], eidx_v)
        pltpu.sync_copy(nidx_hbm.at[pl.ds(base, chunk)], nidx_v)

        def body(i, carry):
            off = i * BLK
            cw = pltpu.async_copy(wtab.at[widx_v.at[pl.ds(off, BLK)]], wbuf, gsem)
            ce = pltpu.async_copy(etab.at[eidx_v.at[pl.ds(off, BLK)]], ebuf, gsem)
            cn = pltpu.async_copy(ntab.at[nidx_v.at[pl.ds(off, BLK)]], nbuf, gsem)
            cw.wait()
            ce.wait()
            cn.wait()
            row = base + off
            pltpu.sync_copy(wbuf, out_hbm.at[pl.ds(row, BLK), pl.ds(0, WORD_DIM)])
            pltpu.sync_copy(ebuf, out_hbm.at[pl.ds(row, BLK), pl.ds(WORD_DIM, DIST_DIM)])
            pltpu.sync_copy(nbuf, out_hbm.at[pl.ds(row, BLK), pl.ds(WORD_DIM + DIST_DIM, DIST_DIM)])
            return carry

        lax.fori_loop(0, nblk, body, 0)

    return k


def kernel(sents, entdists, numdists, word_table, ent_table, num_table):
    B, L = sents.shape
    N = B * L
    widx = sents.reshape(N).astype(jnp.int32)
    eidx = entdists.reshape(N).astype(jnp.int32)
    nidx = numdists.reshape(N).astype(jnp.int32)
    out = _make_sc_kernel(N)(widx, eidx, nidx, word_table, ent_table, num_table)
    return out.reshape(B, L, OUT_DIM)


# speedup vs baseline: 7.8139x; 1.0634x over previous
import functools

import jax
import jax.numpy as jnp
from jax import lax
from jax.experimental import pallas as pl
from jax.experimental.pallas import tpu as pltpu
from jax.experimental.pallas import tpu_sc as plsc

WORD_DIM = 64
DIST_DIM = 32
OUT_DIM = WORD_DIM + 2 * DIST_DIM
BLK = 400


@functools.lru_cache(maxsize=None)
def _make_sc_kernel(N: int):
    info = plsc.get_sparse_core_info()
    NC, NS = info.num_cores, info.num_subcores
    NW = NC * NS
    assert N % (NW * BLK) == 0
    chunk = N // NW
    nblk = chunk // BLK
    mesh = plsc.VectorSubcoreMesh(core_axis_name="c", subcore_axis_name="s")

    buf_types = []
    for _ in range(2):
        buf_types += [
            pltpu.VMEM((BLK, WORD_DIM), jnp.float32),
            pltpu.VMEM((BLK, DIST_DIM), jnp.float32),
            pltpu.VMEM((BLK, DIST_DIM), jnp.float32),
            pltpu.SemaphoreType.DMA,
            pltpu.SemaphoreType.DMA,
        ]

    @functools.partial(
        pl.kernel,
        mesh=mesh,
        out_type=jax.ShapeDtypeStruct((N, OUT_DIM), jnp.float32),
        compiler_params=pltpu.CompilerParams(use_tc_tiling_on_sc=False),
        scratch_types=[
            pltpu.VMEM((chunk,), jnp.int32),
            pltpu.VMEM((chunk,), jnp.int32),
            pltpu.VMEM((chunk,), jnp.int32),
        ] + buf_types,
    )
    def k(widx_hbm, eidx_hbm, nidx_hbm, wtab, etab, ntab, out_hbm,
          widx_v, eidx_v, nidx_v, *bufs):
        sets = [bufs[5 * d:5 * d + 5] for d in range(2)]
        wid = lax.axis_index("s") * NC + lax.axis_index("c")
        base = wid * chunk
        pltpu.sync_copy(widx_hbm.at[pl.ds(base, chunk)], widx_v)
        pltpu.sync_copy(eidx_hbm.at[pl.ds(base, chunk)], eidx_v)
        pltpu.sync_copy(nidx_hbm.at[pl.ds(base, chunk)], nidx_v)

        def fire_gathers(b, s):
            wbuf, ebuf, nbuf, gsem, _ = s
            off = b * BLK
            return [
                pltpu.async_copy(wtab.at[widx_v.at[pl.ds(off, BLK)]], wbuf, gsem),
                pltpu.async_copy(etab.at[eidx_v.at[pl.ds(off, BLK)]], ebuf, gsem),
                pltpu.async_copy(ntab.at[nidx_v.at[pl.ds(off, BLK)]], nbuf, gsem),
            ]

        def fire_writes(b, s):
            wbuf, ebuf, nbuf, _, wsem = s
            row = base + b * BLK
            return [
                pltpu.async_copy(
                    wbuf, out_hbm.at[pl.ds(row, BLK), pl.ds(0, WORD_DIM)], wsem),
                pltpu.async_copy(
                    ebuf, out_hbm.at[pl.ds(row, BLK), pl.ds(WORD_DIM, DIST_DIM)], wsem),
                pltpu.async_copy(
                    nbuf, out_hbm.at[pl.ds(row, BLK), pl.ds(WORD_DIM + DIST_DIM, DIST_DIM)], wsem),
            ]

        gh = {}
        wh = {}
        for b in range(nblk + 1):
            if b < nblk:
                if b >= 2:
                    for h in wh[b - 2]:
                        h.wait()
                gh[b] = fire_gathers(b, sets[b % 2])
            if b >= 1:
                bb = b - 1
                for h in gh[bb]:
                    h.wait()
                wh[bb] = fire_writes(bb, sets[bb % 2])
        for h in wh[nblk - 2] + wh[nblk - 1]:
            h.wait()

    return k


def kernel(sents, entdists, numdists, word_table, ent_table, num_table):
    B, L = sents.shape
    N = B * L
    widx = sents.reshape(N).astype(jnp.int32)
    eidx = entdists.reshape(N).astype(jnp.int32)
    nidx = numdists.reshape(N).astype(jnp.int32)
    out = _make_sc_kernel(N)(widx, eidx, nidx, word_table, ent_table, num_table)
    return out.reshape(B, L, OUT_DIM)


# speedup vs baseline: 10.7480x; 1.3755x over previous
import functools

import jax
import jax.numpy as jnp
from jax import lax
from jax.experimental import pallas as pl
from jax.experimental.pallas import tpu as pltpu
from jax.experimental.pallas import tpu_sc as plsc

WORD_DIM = 64
DIST_DIM = 32
OUT_DIM = WORD_DIM + 2 * DIST_DIM
LPAD = 56
BPB = 8


@functools.lru_cache(maxsize=None)
def _make_sc_kernel(B: int, L: int):
    info = plsc.get_sparse_core_info()
    NC, NS = info.num_cores, info.num_subcores
    NW = NC * NS
    assert B % (NW * BPB) == 0
    BLK = BPB * L
    bchunk = B // NW
    chunk = bchunk * L
    nblk = bchunk // BPB
    N = B * L
    mesh = plsc.VectorSubcoreMesh(core_axis_name="c", subcore_axis_name="s")

    buf_types = []
    for _ in range(2):
        buf_types += [
            pltpu.VMEM((BLK, WORD_DIM), jnp.float32),
            pltpu.VMEM((BLK, DIST_DIM), jnp.float32),
            pltpu.VMEM((BLK, DIST_DIM), jnp.float32),
            pltpu.SemaphoreType.DMA,
            pltpu.SemaphoreType.DMA,
        ]

    @functools.partial(
        pl.kernel,
        mesh=mesh,
        out_type=jax.ShapeDtypeStruct((B, LPAD, OUT_DIM), jnp.float32),
        compiler_params=pltpu.CompilerParams(use_tc_tiling_on_sc=False),
        scratch_types=[
            pltpu.VMEM((chunk,), jnp.int32),
            pltpu.VMEM((chunk,), jnp.int32),
            pltpu.VMEM((chunk,), jnp.int32),
        ] + buf_types,
    )
    def k(widx_hbm, eidx_hbm, nidx_hbm, wtab, etab, ntab, out_hbm,
          widx_v, eidx_v, nidx_v, *bufs):
        sets = [bufs[5 * d:5 * d + 5] for d in range(2)]
        wid = lax.axis_index("s") * NC + lax.axis_index("c")
        base = wid * chunk
        bbase = wid * bchunk
        pltpu.sync_copy(widx_hbm.at[pl.ds(base, chunk)], widx_v)
        pltpu.sync_copy(eidx_hbm.at[pl.ds(base, chunk)], eidx_v)
        pltpu.sync_copy(nidx_hbm.at[pl.ds(base, chunk)], nidx_v)

        def fire_gathers(b, s):
            wbuf, ebuf, nbuf, gsem, _ = s
            off = b * BLK
            return [
                pltpu.async_copy(wtab.at[widx_v.at[pl.ds(off, BLK)]], wbuf, gsem),
                pltpu.async_copy(etab.at[eidx_v.at[pl.ds(off, BLK)]], ebuf, gsem),
                pltpu.async_copy(ntab.at[nidx_v.at[pl.ds(off, BLK)]], nbuf, gsem),
            ]

        def fire_writes(b, s):
            wbuf, ebuf, nbuf, _, wsem = s
            bat = bbase + b * BPB
            hs = []
            for j in range(BPB):
                row = out_hbm.at[bat + j]
                hs += [
                    pltpu.async_copy(
                        wbuf.at[pl.ds(j * L, L)],
                        row.at[pl.ds(0, L), pl.ds(0, WORD_DIM)], wsem),
                    pltpu.async_copy(
                        ebuf.at[pl.ds(j * L, L)],
                        row.at[pl.ds(0, L), pl.ds(WORD_DIM, DIST_DIM)], wsem),
                    pltpu.async_copy(
                        nbuf.at[pl.ds(j * L, L)],
                        row.at[pl.ds(0, L), pl.ds(WORD_DIM + DIST_DIM, DIST_DIM)], wsem),
                ]
            return hs

        gh = {}
        wh = {}
        for b in range(nblk + 1):
            if b < nblk:
                if b >= 2:
                    for h in wh[b - 2]:
                        h.wait()
                gh[b] = fire_gathers(b, sets[b % 2])
            if b >= 1:
                bb = b - 1
                for h in gh[bb]:
                    h.wait()
                wh[bb] = fire_writes(bb, sets[bb % 2])
        for h in wh[nblk - 2] + wh[nblk - 1]:
            h.wait()

    return k


def kernel(sents, entdists, numdists, word_table, ent_table, num_table):
    B, L = sents.shape
    N = B * L
    widx = sents.reshape(N).astype(jnp.int32)
    eidx = entdists.reshape(N).astype(jnp.int32)
    nidx = numdists.reshape(N).astype(jnp.int32)
    out = _make_sc_kernel(B, L)(widx, eidx, nidx, word_table, ent_table, num_table)
    return out[:, :L, :]
